# Initial kernel scaffold; baseline (speedup 1.0000x reference)
import jax
import jax.numpy as jnp
from jax.experimental import pallas as pl


def kernel(node_embedding, gumbel_u, V_idx, E_idx):
    raise NotImplementedError("write your pallas kernel here")


# speedup vs baseline: 3.0868x; 3.0868x over previous
import functools

import jax
import jax.numpy as jnp
from jax import lax
from jax.experimental import pallas as pl
from jax.experimental.pallas import tpu as pltpu
from jax.experimental.pallas import tpu_sc as plsc

EPS = 1e-10
V = 50000
E = 25000
NNZ = 800000
S = 8
D = 8
SD = S * D
TAU = 1.0

NC = 2
NS = 16
CHUNK = 128
NCHUNKS = NNZ // CHUNK
ACC_R = 25600
TILE_R = ACC_R // NS
HALF = V // 2
TRASH = HALF


def _zero_fill(ref, ncols):
    def body(i, _):
        for k in range(ncols // 16):
            ref[i, pl.ds(k * 16, 16)] = jnp.zeros((16,), jnp.float32)
        return 0
    lax.fori_loop(0, 128, body, 0)


def _ones_fill(ref):
    def body(i, _):
        ref[i, pl.ds(0, 16)] = jnp.ones((16,), jnp.float32)
        return 0
    lax.fori_loop(0, 128, body, 0)


def _init_spmem(acc, zbuf, tile):
    row0 = tile * TILE_R
    nfull = TILE_R // 128
    rem = TILE_R - nfull * 128
    for k in range(nfull):
        pltpu.sync_copy(zbuf, acc.at[pl.ds(row0 + k * 128, 128)])
    if rem:
        pltpu.sync_copy(zbuf.at[pl.ds(0, rem)], acc.at[pl.ds(row0 + nfull * 128, rem)])


def _dump_spmem(acc, out_hbm, core, tile):
    row0 = tile * TILE_R
    out0 = core * ACC_R + row0
    nfull = TILE_R // 128
    rem = TILE_R - nfull * 128
    for k in range(nfull):
        pltpu.sync_copy(acc.at[pl.ds(row0 + k * 128, 128)],
                        out_hbm.at[pl.ds(out0 + k * 128, 128)])
    if rem:
        pltpu.sync_copy(acc.at[pl.ds(row0 + nfull * 128, rem)],
                        out_hbm.at[pl.ds(out0 + nfull * 128, rem)])


def _remap_half(src_v, dst_v, core):
    lo = core * HALF

    def remap(kk, _c):
        sl = pl.ds(kk * 16, 16)
        loc = src_v[sl] - lo
        inr = (loc >= 0) & (loc < HALF)
        dst_v[sl] = jnp.where(inr, loc, TRASH)
        return 0
    lax.fori_loop(0, CHUNK // 16, remap, 0)


def _sc_counts(vidx, eidx):
    mesh = plsc.VectorSubcoreMesh(core_axis_name="c", subcore_axis_name="s")

    @functools.partial(
        pl.kernel,
        out_type=(
            jax.ShapeDtypeStruct((NC * ACC_R, 16), jnp.float32),
            jax.ShapeDtypeStruct((NC * ACC_R, 16), jnp.float32),
        ),
        mesh=mesh,
        compiler_params=pltpu.CompilerParams(use_tc_tiling_on_sc=False),
        scratch_types=[
            pltpu.VMEM((CHUNK,), jnp.int32),
            pltpu.VMEM((CHUNK,), jnp.int32),
            pltpu.VMEM((CHUNK, 16), jnp.float32),
            pltpu.VMEM((CHUNK, 16), jnp.float32),
            pltpu.VMEM_SHARED((ACC_R, 16), jnp.float32),
            pltpu.VMEM_SHARED((ACC_R, 16), jnp.float32),
        ],
    )
    def k(vidx_hbm, eidx_hbm, cnte_hbm, cntv_hbm,
          ei_v, sm_v, ones_v, z16_v, ce_sh, cv_sh):
        core = lax.axis_index("c")
        tile = lax.axis_index("s")
        _zero_fill(z16_v, 16)
        _ones_fill(ones_v)
        _init_spmem(ce_sh, z16_v, tile)
        _init_spmem(cv_sh, z16_v, tile)
        plsc.subcore_barrier()

        def body_e(j, _):
            cid = core * NS + tile + j * (NC * NS)

            @pl.when(cid < NCHUNKS)
            def _():
                pltpu.sync_copy(eidx_hbm.at[pl.ds(cid * CHUNK, CHUNK)], ei_v)
                pltpu.sync_copy(ones_v, ce_sh.at[ei_v], add=True)
            return 0

        def body_v(j, _):
            cid = tile + j * NS

            @pl.when(cid < NCHUNKS)
            def _():
                pltpu.sync_copy(vidx_hbm.at[pl.ds(cid * CHUNK, CHUNK)], ei_v)
                _remap_half(ei_v, sm_v, core)
                pltpu.sync_copy(ones_v, cv_sh.at[sm_v], add=True)
            return 0

        lax.fori_loop(0, NCHUNKS // (NC * NS) + 1, body_e, 0)
        lax.fori_loop(0, NCHUNKS // NS + 1, body_v, 0)
        plsc.subcore_barrier()
        _dump_spmem(ce_sh, cnte_hbm, core, tile)
        _dump_spmem(cv_sh, cntv_hbm, core, tile)

    return k(vidx, eidx)


def _sc_segment_sum(table, gidx, sidx, split_by_half):
    mesh = plsc.VectorSubcoreMesh(core_axis_name="c", subcore_axis_name="s")

    @functools.partial(
        pl.kernel,
        out_type=jax.ShapeDtypeStruct((NC * ACC_R, SD), jnp.float32),
        mesh=mesh,
        compiler_params=pltpu.CompilerParams(use_tc_tiling_on_sc=False),
        scratch_types=[
            pltpu.VMEM((CHUNK,), jnp.int32),
            pltpu.VMEM((CHUNK,), jnp.int32),
            pltpu.VMEM((CHUNK,), jnp.int32),
            pltpu.VMEM((CHUNK, SD), jnp.float32),
            pltpu.VMEM((CHUNK, SD), jnp.float32),
            pltpu.VMEM_SHARED((ACC_R, SD), jnp.float32),
            pltpu.SemaphoreType.DMA,
        ],
    )
    def k(table_hbm, gidx_hbm, sidx_hbm, sum_hbm,
          gi_v, si_v, sm_v, rows_v, z64_v, acc_sh, sem):
        core = lax.axis_index("c")
        tile = lax.axis_index("s")

        _zero_fill(z64_v, SD)
        _init_spmem(acc_sh, z64_v, tile)
        plsc.subcore_barrier()

        if split_by_half:
            ntrip = NCHUNKS // NS + 1
            stride = NS
            first = tile
        else:
            ntrip = NCHUNKS // (NC * NS) + 1
            stride = NC * NS
            first = core * NS + tile

        def body(j, _):
            cid = first + j * stride

            @pl.when(cid < NCHUNKS)
            def _():
                base = cid * CHUNK
                pltpu.sync_copy(gidx_hbm.at[pl.ds(base, CHUNK)], gi_v)
                pltpu.sync_copy(sidx_hbm.at[pl.ds(base, CHUNK)], si_v)
                if split_by_half:
                    _remap_half(si_v, sm_v, core)
                    scatter_idx = sm_v
                else:
                    scatter_idx = si_v
                pltpu.async_copy(table_hbm.at[gi_v], rows_v, sem).wait()
                pltpu.sync_copy(rows_v, acc_sh.at[scatter_idx], add=True)
            return 0

        lax.fori_loop(0, ntrip, body, 0)
        plsc.subcore_barrier()
        _dump_spmem(acc_sh, sum_hbm, core, tile)

    return k(table, gidx, sidx)


def _tc_gumbel_softmax(ne, gu):
    BR = 400

    def body(ne_ref, gu_ref, out_ref):
        g = -jnp.log(-jnp.log(gu_ref[...] + EPS) + EPS)
        x = (ne_ref[...] + g) / TAU
        for grp in range(S):
            sl = slice(grp * D, (grp + 1) * D)
            xg = x[:, sl]
            m = jnp.max(xg, axis=1, keepdims=True)
            e = jnp.exp(xg - m)
            out_ref[:, sl] = e / jnp.sum(e, axis=1, keepdims=True)

    return pl.pallas_call(
        body,
        grid=(V // BR,),
        in_specs=[
            pl.BlockSpec((BR, SD), lambda i: (i, 0)),
            pl.BlockSpec((BR, SD), lambda i: (i, 0)),
        ],
        out_specs=pl.BlockSpec((BR, SD), lambda i: (i, 0)),
        out_shape=jax.ShapeDtypeStruct((V, SD), jnp.float32),
    )(ne, gu)


def _stats_body(x, ent_ref, psum_ref, n2_ref, g_ref, first):
    @pl.when(first)
    def _():
        ent_ref[...] = jnp.zeros_like(ent_ref)
        psum_ref[...] = jnp.zeros_like(psum_ref)
        n2_ref[...] = jnp.zeros_like(n2_ref)
        g_ref[...] = jnp.zeros_like(g_ref)

    ent_ref[...] += jnp.sum(-x * jnp.log(x + EPS))
    psum_ref[...] += jnp.sum(x, axis=0, keepdims=True)
    n2_ref[...] += jnp.sum(x * x, axis=0, keepdims=True)
    for s in range(S):
        xs = x[:, s * D:(s + 1) * D]
        g_ref[s * D:(s + 1) * D, :] += lax.dot_general(
            xs, xs, (((0,), (0,)), ((), ())),
            preferred_element_type=jnp.float32)


def _tc_stats_y(psum, pcnt):
    BR = 200
    nblk = E // BR
    off = ACC_R // BR

    def body(p0_ref, p1_ref, c0_ref, c1_ref,
             yj_ref, ent_ref, psum_ref, n2_ref, g_ref):
        i = pl.program_id(0)
        cnt = c0_ref[:, 0:1] + c1_ref[:, 0:1]
        yj = (p0_ref[...] + p1_ref[...]) / jnp.maximum(cnt, 1.0)
        yj_ref[...] = yj
        _stats_body(yj, ent_ref, psum_ref, n2_ref, g_ref, i == 0)

    return pl.pallas_call(
        body,
        grid=(nblk,),
        in_specs=[
            pl.BlockSpec((BR, SD), lambda i: (i, 0)),
            pl.BlockSpec((BR, SD), lambda i: (i + off, 0)),
            pl.BlockSpec((BR, 16), lambda i: (i, 0)),
            pl.BlockSpec((BR, 16), lambda i: (i + off, 0)),
        ],
        out_specs=[
            pl.BlockSpec((BR, SD), lambda i: (i, 0)),
            pl.BlockSpec((1, 1), lambda i: (0, 0)),
            pl.BlockSpec((1, SD), lambda i: (0, 0)),
            pl.BlockSpec((1, SD), lambda i: (0, 0)),
            pl.BlockSpec((SD, D), lambda i: (0, 0)),
        ],
        out_shape=[
            jax.ShapeDtypeStruct((E, SD), jnp.float32),
            jax.ShapeDtypeStruct((1, 1), jnp.float32),
            jax.ShapeDtypeStruct((1, SD), jnp.float32),
            jax.ShapeDtypeStruct((1, SD), jnp.float32),
            jax.ShapeDtypeStruct((SD, D), jnp.float32),
        ],
    )(psum, psum, pcnt, pcnt)


def _tc_stats_x(psum, pcnt):
    BR = 200
    nblk = V // BR
    skip = ACC_R // BR
    half_blk = HALF // BR

    def rowmap(i):
        return (jnp.where(i < half_blk, i, i + (skip - half_blk)), 0)

    def body(p_ref, c_ref, ent_ref, psum_ref, n2_ref, g_ref):
        i = pl.program_id(0)
        xj = p_ref[...] / jnp.maximum(c_ref[:, 0:1], 1.0)
        _stats_body(xj, ent_ref, psum_ref, n2_ref, g_ref, i == 0)

    return pl.pallas_call(
        body,
        grid=(nblk,),
        in_specs=[
            pl.BlockSpec((BR, SD), rowmap),
            pl.BlockSpec((BR, 16), rowmap),
        ],
        out_specs=[
            pl.BlockSpec((1, 1), lambda i: (0, 0)),
            pl.BlockSpec((1, SD), lambda i: (0, 0)),
            pl.BlockSpec((1, SD), lambda i: (0, 0)),
            pl.BlockSpec((SD, D), lambda i: (0, 0)),
        ],
        out_shape=[
            jax.ShapeDtypeStruct((1, 1), jnp.float32),
            jax.ShapeDtypeStruct((1, SD), jnp.float32),
            jax.ShapeDtypeStruct((1, SD), jnp.float32),
            jax.ShapeDtypeStruct((SD, D), jnp.float32),
        ],
    )(psum, pcnt)


def _finish(ent, psum, n2, g, n_rows):
    ent_mean = ent[0, 0] / (n_rows * S)
    p = psum.reshape(S, D) / n_rows
    glob_ent = -jnp.mean(-jnp.sum(p * jnp.log(p + EPS), axis=1))
    norms = jnp.sqrt(n2.reshape(S, D))
    gm = g.reshape(S, D, D)
    den = jnp.maximum(norms, EPS)
    c = gm / (den[:, :, None] * den[:, None, :])
    c = jax.nn.softmax(c, axis=2)
    diag = jnp.diagonal(c, axis1=1, axis2=2)
    disc = jnp.mean(-jnp.log(diag))
    return ent_mean, glob_ent + disc


def kernel(node_embedding, gumbel_u, V_idx, E_idx):
    gu = gumbel_u.reshape(V, SD)
    x0 = _tc_gumbel_softmax(node_embedding, gu)
    ycnt, xcnt = _sc_counts(V_idx, E_idx)

    ysum = _sc_segment_sum(x0, V_idx, E_idx, split_by_half=False)
    yj, ent_y, psum_y, n2_y, g_y = _tc_stats_y(ysum, ycnt)

    xsum = _sc_segment_sum(yj, E_idx, V_idx, split_by_half=True)
    ent_x, psum_x, n2_x, g_x = _tc_stats_x(xsum, xcnt)

    ly, gy = _finish(ent_y, psum_y, n2_y, g_y, E)
    lx, gx = _finish(ent_x, psum_x, n2_x, g_x, V)
    return (ly + lx, gy + gx)


# speedup vs baseline: 3.9580x; 1.2822x over previous
import functools

import jax
import jax.numpy as jnp
from jax import lax
from jax.experimental import pallas as pl
from jax.experimental.pallas import tpu as pltpu
from jax.experimental.pallas import tpu_sc as plsc

EPS = 1e-10
V = 50000
E = 25000
NNZ = 800000
S = 8
D = 8
SD = S * D
TAU = 1.0

NC = 2
NS = 16
NW = NC * NS
CHUNK = 128
WCH = 196
NCH = NW * WCH
NNZ_PAD = NCH * CHUNK
ACC_R = 25600
TILE_R = ACC_R // NS
HALF = V // 2
TRASH = HALF
XP_R = 50176
YP_R = 25200
NB = 2
SS = 14


def _zero_fill(ref, ncols):
    def body(i, _):
        for k in range(ncols // 16):
            ref[i, pl.ds(k * 16, 16)] = jnp.zeros((16,), jnp.float32)
        return 0
    lax.fori_loop(0, 128, body, 0)


def _ones_fill(ref):
    def body(i, _):
        ref[i, pl.ds(0, 16)] = jnp.ones((16,), jnp.float32)
        return 0
    lax.fori_loop(0, 128, body, 0)


def _init_spmem(acc, zbuf, tile):
    row0 = tile * TILE_R
    nfull = TILE_R // 128
    rem = TILE_R - nfull * 128
    for k in range(nfull):
        pltpu.sync_copy(zbuf, acc.at[pl.ds(row0 + k * 128, 128)])
    if rem:
        pltpu.sync_copy(zbuf.at[pl.ds(0, rem)], acc.at[pl.ds(row0 + nfull * 128, rem)])


def _dump_spmem(acc, out_hbm, core, tile):
    row0 = tile * TILE_R
    out0 = core * ACC_R + row0
    nfull = TILE_R // 128
    rem = TILE_R - nfull * 128
    for k in range(nfull):
        pltpu.sync_copy(acc.at[pl.ds(row0 + k * 128, 128)],
                        out_hbm.at[pl.ds(out0 + k * 128, 128)])
    if rem:
        pltpu.sync_copy(acc.at[pl.ds(row0 + nfull * 128, rem)],
                        out_hbm.at[pl.ds(out0 + nfull * 128, rem)])


def _remap_slab(idx_slab, core, nrows):
    lo = core * HALF

    def body(j, _):
        for k in range(CHUNK // 16):
            sl = pl.ds(k * 16, 16)
            loc = idx_slab[j, sl] - lo
            inr = (loc >= 0) & (loc < HALF)
            idx_slab[j, sl] = jnp.where(inr, loc, TRASH)
        return 0
    lax.fori_loop(0, nrows, body, 0)


def _sc_counts(vidx2d, eidx2d):
    mesh = plsc.VectorSubcoreMesh(core_axis_name="c", subcore_axis_name="s")

    @functools.partial(
        pl.kernel,
        out_type=(
            jax.ShapeDtypeStruct((NC * ACC_R, 16), jnp.float32),
            jax.ShapeDtypeStruct((NC * ACC_R, 16), jnp.float32),
        ),
        mesh=mesh,
        compiler_params=pltpu.CompilerParams(use_tc_tiling_on_sc=False),
        scratch_types=[
            pltpu.VMEM((WCH, CHUNK), jnp.int32),
            pltpu.VMEM((WCH, CHUNK), jnp.int32),
            pltpu.VMEM((128, 16), jnp.float32),
            pltpu.VMEM((128, 16), jnp.float32),
            pltpu.VMEM_SHARED((ACC_R, 16), jnp.float32),
            pltpu.VMEM_SHARED((ACC_R, 16), jnp.float32),
            pltpu.SemaphoreType.DMA,
        ],
    )
    def k(vidx_hbm, eidx_hbm, cnte_hbm, cntv_hbm,
          ia_v, ib_v, ones_v, z16_v, ce_sh, cv_sh, sem):
        core = lax.axis_index("c")
        tile = lax.axis_index("s")
        _zero_fill(z16_v, 16)
        _ones_fill(ones_v)
        _init_spmem(ce_sh, z16_v, tile)
        _init_spmem(cv_sh, z16_v, tile)
        plsc.subcore_barrier()

        def fire(idx_slab, acc):
            def body(j, _):
                pltpu.async_copy(ones_v, acc.at[idx_slab.at[j]], sem, add=True)
                return 0
            lax.fori_loop(0, WCH, body, 0)

        def drain(n, acc):
            def body(j, _):
                pltpu.make_async_copy(ones_v, acc.at[ia_v.at[0]], sem).wait()
                return 0
            lax.fori_loop(0, n, body, 0)

        w = core * NS + tile
        pltpu.sync_copy(eidx_hbm.at[pl.ds(w * WCH, WCH)], ia_v)
        fire(ia_v, ce_sh)
        pltpu.sync_copy(vidx_hbm.at[pl.ds(tile * 2 * WCH, WCH)], ib_v)
        _remap_slab(ib_v, core, WCH)
        fire(ib_v, cv_sh)
        drain(2 * WCH, ce_sh)
        pltpu.sync_copy(vidx_hbm.at[pl.ds(tile * 2 * WCH + WCH, WCH)], ia_v)
        _remap_slab(ia_v, core, WCH)
        fire(ia_v, cv_sh)
        drain(WCH, cv_sh)
        plsc.subcore_barrier()
        _dump_spmem(ce_sh, cnte_hbm, core, tile)
        _dump_spmem(cv_sh, cntv_hbm, core, tile)

    return k(vidx2d, eidx2d)


def _sc_segment_sum(table, gidx2d, sidx2d, split_by_half):
    mesh = plsc.VectorSubcoreMesh(core_axis_name="c", subcore_axis_name="s")
    nslab = 2 if split_by_half else 1

    @functools.partial(
        pl.kernel,
        out_type=jax.ShapeDtypeStruct((NC * ACC_R, SD), jnp.float32),
        mesh=mesh,
        compiler_params=pltpu.CompilerParams(use_tc_tiling_on_sc=False),
        scratch_types=[
            pltpu.VMEM((SS, CHUNK), jnp.int32),
            pltpu.VMEM((SS, CHUNK), jnp.int32),
            [pltpu.VMEM((CHUNK, SD), jnp.float32)] * NB,
            pltpu.VMEM_SHARED((ACC_R, SD), jnp.float32),
            pltpu.SemaphoreType.DMA,
            pltpu.SemaphoreType.DMA,
        ],
    )
    def k(table_hbm, gidx_hbm, sidx_hbm, sum_hbm,
          gi_v, si_v, bufs, acc_sh, semg, sems):
        core = lax.axis_index("c")
        tile = lax.axis_index("s")

        _zero_fill(bufs[0], SD)
        _init_spmem(acc_sh, bufs[0], tile)
        plsc.subcore_barrier()

        def gather(j, b):
            pltpu.async_copy(table_hbm.at[gi_v.at[j]], bufs[b], semg)

        def wait_g(j, b):
            pltpu.make_async_copy(table_hbm.at[gi_v.at[j]], bufs[b], semg).wait()

        def scat(j, b):
            pltpu.async_copy(bufs[b], acc_sh.at[si_v.at[j]], sems, add=True)

        def wait_s(j, b):
            pltpu.make_async_copy(bufs[b], acc_sh.at[si_v.at[j]], sems).wait()

        if split_by_half:
            base0 = tile * nslab * WCH
        else:
            base0 = (core * NS + tile) * WCH
        nsub = nslab * WCH // SS

        def sub_slab(s, _):
            base = base0 + s * SS
            pltpu.sync_copy(gidx_hbm.at[pl.ds(base, SS)], gi_v)
            pltpu.sync_copy(sidx_hbm.at[pl.ds(base, SS)], si_v)
            if split_by_half:
                _remap_slab(si_v, core, SS)

            gather(0, 0)
            for j in range(SS):
                if j + 1 < SS:
                    if j >= 1:
                        wait_s(j - 1, (j + 1) % NB)
                    gather(j + 1, (j + 1) % NB)
                wait_g(j, j % NB)
                scat(j, j % NB)
            wait_s(SS - 2, (SS - 2) % NB)
            wait_s(SS - 1, (SS - 1) % NB)
            return 0

        lax.fori_loop(0, nsub, sub_slab, 0)
        plsc.subcore_barrier()
        _dump_spmem(acc_sh, sum_hbm, core, tile)

    return k(table, gidx2d, sidx2d)


def _tc_gumbel_softmax(ne, gu):
    BR = 400

    def body(ne_ref, gu_ref, out_ref):
        g = -jnp.log(-jnp.log(gu_ref[...] + EPS) + EPS)
        x = (ne_ref[...] + g) / TAU
        for grp in range(S):
            sl = slice(grp * D, (grp + 1) * D)
            xg = x[:, sl]
            m = jnp.max(xg, axis=1, keepdims=True)
            e = jnp.exp(xg - m)
            out_ref[:, sl] = e / jnp.sum(e, axis=1, keepdims=True)

    return pl.pallas_call(
        body,
        grid=(V // BR,),
        in_specs=[
            pl.BlockSpec((BR, SD), lambda i: (i, 0)),
            pl.BlockSpec((BR, SD), lambda i: (i, 0)),
        ],
        out_specs=pl.BlockSpec((BR, SD), lambda i: (i, 0)),
        out_shape=jax.ShapeDtypeStruct((V, SD), jnp.float32),
    )(ne, gu)


def _stats_body(x, ent_ref, psum_ref, n2_ref, g_ref, first):
    @pl.when(first)
    def _():
        ent_ref[...] = jnp.zeros_like(ent_ref)
        psum_ref[...] = jnp.zeros_like(psum_ref)
        n2_ref[...] = jnp.zeros_like(n2_ref)
        g_ref[...] = jnp.zeros_like(g_ref)

    ent_ref[...] += jnp.sum(-x * jnp.log(x + EPS))
    psum_ref[...] += jnp.sum(x, axis=0, keepdims=True)
    n2_ref[...] += jnp.sum(x * x, axis=0, keepdims=True)
    for s in range(S):
        xs = x[:, s * D:(s + 1) * D]
        g_ref[s * D:(s + 1) * D, :] += lax.dot_general(
            xs, xs, (((0,), (0,)), ((), ())),
            preferred_element_type=jnp.float32)


def _tc_stats_y(psum, pcnt):
    BR = 200
    nblk = E // BR
    off = ACC_R // BR

    def body(p0_ref, p1_ref, c0_ref, c1_ref,
             yj_ref, ent_ref, psum_ref, n2_ref, g_ref):
        i = pl.program_id(0)
        cnt = c0_ref[:, 0:1] + c1_ref[:, 0:1]
        yj = (p0_ref[...] + p1_ref[...]) / jnp.maximum(cnt, 1.0)
        yj_ref[...] = yj

        @pl.when(i < nblk)
        def _():
            _stats_body(yj, ent_ref, psum_ref, n2_ref, g_ref, i == 0)

    return pl.pallas_call(
        body,
        grid=(nblk + 1,),
        in_specs=[
            pl.BlockSpec((BR, SD), lambda i: (jnp.minimum(i, nblk - 1), 0)),
            pl.BlockSpec((BR, SD), lambda i: (jnp.minimum(i, nblk - 1) + off, 0)),
            pl.BlockSpec((BR, 16), lambda i: (jnp.minimum(i, nblk - 1), 0)),
            pl.BlockSpec((BR, 16), lambda i: (jnp.minimum(i, nblk - 1) + off, 0)),
        ],
        out_specs=[
            pl.BlockSpec((BR, SD), lambda i: (i, 0)),
            pl.BlockSpec((1, 1), lambda i: (0, 0)),
            pl.BlockSpec((1, SD), lambda i: (0, 0)),
            pl.BlockSpec((1, SD), lambda i: (0, 0)),
            pl.BlockSpec((SD, D), lambda i: (0, 0)),
        ],
        out_shape=[
            jax.ShapeDtypeStruct((YP_R, SD), jnp.float32),
            jax.ShapeDtypeStruct((1, 1), jnp.float32),
            jax.ShapeDtypeStruct((1, SD), jnp.float32),
            jax.ShapeDtypeStruct((1, SD), jnp.float32),
            jax.ShapeDtypeStruct((SD, D), jnp.float32),
        ],
    )(psum, psum, pcnt, pcnt)


def _tc_stats_x(psum, pcnt):
    BR = 200
    nblk = V // BR
    skip = ACC_R // BR
    half_blk = HALF // BR

    def rowmap(i):
        return (jnp.where(i < half_blk, i, i + (skip - half_blk)), 0)

    def body(p_ref, c_ref, ent_ref, psum_ref, n2_ref, g_ref):
        i = pl.program_id(0)
        xj = p_ref[...] / jnp.maximum(c_ref[:, 0:1], 1.0)
        _stats_body(xj, ent_ref, psum_ref, n2_ref, g_ref, i == 0)

    return pl.pallas_call(
        body,
        grid=(nblk,),
        in_specs=[
            pl.BlockSpec((BR, SD), rowmap),
            pl.BlockSpec((BR, 16), rowmap),
        ],
        out_specs=[
            pl.BlockSpec((1, 1), lambda i: (0, 0)),
            pl.BlockSpec((1, SD), lambda i: (0, 0)),
            pl.BlockSpec((1, SD), lambda i: (0, 0)),
            pl.BlockSpec((SD, D), lambda i: (0, 0)),
        ],
        out_shape=[
            jax.ShapeDtypeStruct((1, 1), jnp.float32),
            jax.ShapeDtypeStruct((1, SD), jnp.float32),
            jax.ShapeDtypeStruct((1, SD), jnp.float32),
            jax.ShapeDtypeStruct((SD, D), jnp.float32),
        ],
    )(psum, pcnt)


def _finish(ent, psum, n2, g, n_rows):
    ent_mean = ent[0, 0] / (n_rows * S)
    p = psum.reshape(S, D) / n_rows
    glob_ent = -jnp.mean(-jnp.sum(p * jnp.log(p + EPS), axis=1))
    norms = jnp.sqrt(n2.reshape(S, D))
    gm = g.reshape(S, D, D)
    den = jnp.maximum(norms, EPS)
    c = gm / (den[:, :, None] * den[:, None, :])
    c = jax.nn.softmax(c, axis=2)
    diag = jnp.diagonal(c, axis1=1, axis2=2)
    disc = jnp.mean(-jnp.log(diag))
    return ent_mean, glob_ent + disc


def kernel(node_embedding, gumbel_u, V_idx, E_idx):
    gu = gumbel_u.reshape(V, SD)
    x0 = _tc_gumbel_softmax(node_embedding, gu)
    pad = NNZ_PAD - NNZ
    vidx2d = jnp.concatenate(
        [V_idx, jnp.full((pad,), V, jnp.int32)]).reshape(NCH, CHUNK)
    eidx2d = jnp.concatenate(
        [E_idx, jnp.full((pad,), E, jnp.int32)]).reshape(NCH, CHUNK)
    x0p = jnp.concatenate([x0, jnp.zeros((XP_R - V, SD), jnp.float32)])

    ycnt, xcnt = _sc_counts(vidx2d, eidx2d)

    ysum = _sc_segment_sum(x0p, vidx2d, eidx2d, split_by_half=False)
    yjp, ent_y, psum_y, n2_y, g_y = _tc_stats_y(ysum, ycnt)

    xsum = _sc_segment_sum(yjp, eidx2d, vidx2d, split_by_half=True)
    ent_x, psum_x, n2_x, g_x = _tc_stats_x(xsum, xcnt)

    ly, gy = _finish(ent_y, psum_y, n2_y, g_y, E)
    lx, gx = _finish(ent_x, psum_x, n2_x, g_x, V)
    return (ly + lx, gy + gx)


# speedup vs baseline: 5.0670x; 1.2802x over previous
import functools

import jax
import jax.numpy as jnp
from jax import lax
from jax.experimental import pallas as pl
from jax.experimental.pallas import tpu as pltpu
from jax.experimental.pallas import tpu_sc as plsc

EPS = 1e-10
V = 50000
E = 25000
NNZ = 800000
S = 8
D = 8
SD = S * D
TAU = 1.0

NC = 2
NS = 16
NW = NC * NS
CHUNK = 128
WCH = 196
NCH = NW * WCH
NNZ_PAD = NCH * CHUNK
ACC_R = 25600
TILE_R = ACC_R // NS
HALF = V // 2
TRASH = HALF
XP_R = 50400
YP_R = 25200
VC_R = 51200
NB = 2
SS = 14
SSC = 98


def _zero_fill(ref, ncols):
    def body(i, _):
        for k in range(ncols // 16):
            ref[i, pl.ds(k * 16, 16)] = jnp.zeros((16,), jnp.float32)
        return 0
    lax.fori_loop(0, 128, body, 0)


def _ones_fill(ref):
    def body(i, _):
        ref[i, pl.ds(0, 16)] = jnp.ones((16,), jnp.float32)
        return 0
    lax.fori_loop(0, 128, body, 0)


def _init_spmem(acc, zbuf, tile, tile_rows=TILE_R):
    row0 = tile * tile_rows
    nfull = tile_rows // 128
    rem = tile_rows - nfull * 128
    for k in range(nfull):
        pltpu.sync_copy(zbuf, acc.at[pl.ds(row0 + k * 128, 128)])
    if rem:
        pltpu.sync_copy(zbuf.at[pl.ds(0, rem)], acc.at[pl.ds(row0 + nfull * 128, rem)])


def _dump_spmem(acc, out_hbm, core, tile, tile_rows=TILE_R, acc_rows=ACC_R):
    row0 = tile * tile_rows
    out0 = core * acc_rows + row0
    nfull = tile_rows // 128
    rem = tile_rows - nfull * 128
    for k in range(nfull):
        pltpu.sync_copy(acc.at[pl.ds(row0 + k * 128, 128)],
                        out_hbm.at[pl.ds(out0 + k * 128, 128)])
    if rem:
        pltpu.sync_copy(acc.at[pl.ds(row0 + nfull * 128, rem)],
                        out_hbm.at[pl.ds(out0 + nfull * 128, rem)])


def _remap_slab(idx_slab, core, nrows):
    lo = core * HALF

    def body(j, _):
        for k in range(CHUNK // 16):
            sl = pl.ds(k * 16, 16)
            loc = idx_slab[j, sl] - lo
            inr = (loc >= 0) & (loc < HALF)
            idx_slab[j, sl] = jnp.where(inr, loc, TRASH)
        return 0
    lax.fori_loop(0, nrows, body, 0)


def _sc_counts(vidx2d, eidx2d):
    mesh = plsc.VectorSubcoreMesh(core_axis_name="c", subcore_axis_name="s")

    @functools.partial(
        pl.kernel,
        out_type=(
            jax.ShapeDtypeStruct((NC * ACC_R, 16), jnp.float32),
            jax.ShapeDtypeStruct((NC * VC_R, 16), jnp.float32),
        ),
        mesh=mesh,
        compiler_params=pltpu.CompilerParams(use_tc_tiling_on_sc=False),
        scratch_types=[
            pltpu.VMEM((SSC, CHUNK), jnp.int32),
            pltpu.VMEM((SSC, CHUNK), jnp.int32),
            pltpu.VMEM((128, 16), jnp.float32),
            pltpu.VMEM((128, 16), jnp.float32),
            pltpu.VMEM_SHARED((ACC_R, 16), jnp.float32),
            pltpu.VMEM_SHARED((VC_R, 16), jnp.float32),
            pltpu.SemaphoreType.DMA,
        ],
    )
    def k(vidx_hbm, eidx_hbm, cnte_hbm, cntv_hbm,
          ia_v, ib_v, ones_v, z16_v, ce_sh, cv_sh, sem):
        core = lax.axis_index("c")
        tile = lax.axis_index("s")
        _zero_fill(z16_v, 16)
        _ones_fill(ones_v)
        _init_spmem(ce_sh, z16_v, tile)
        _init_spmem(cv_sh, z16_v, tile, VC_R // NS)
        plsc.subcore_barrier()

        def fire(idx_slab, acc):
            def body(j, _):
                pltpu.async_copy(ones_v, acc.at[idx_slab.at[j]], sem, add=True)
                return 0
            lax.fori_loop(0, SSC, body, 0)

        def drain(n, acc):
            def body(j, _):
                pltpu.make_async_copy(ones_v, acc.at[ia_v.at[0]], sem).wait()
                return 0
            lax.fori_loop(0, n, body, 0)

        w = core * NS + tile
        pltpu.sync_copy(eidx_hbm.at[pl.ds(w * WCH, SSC)], ia_v)
        fire(ia_v, ce_sh)
        pltpu.sync_copy(eidx_hbm.at[pl.ds(w * WCH + SSC, SSC)], ib_v)
        fire(ib_v, ce_sh)
        drain(2 * SSC, ce_sh)
        pltpu.sync_copy(vidx_hbm.at[pl.ds(w * WCH, SSC)], ia_v)
        fire(ia_v, cv_sh)
        pltpu.sync_copy(vidx_hbm.at[pl.ds(w * WCH + SSC, SSC)], ib_v)
        fire(ib_v, cv_sh)
        drain(2 * SSC, cv_sh)
        plsc.subcore_barrier()
        _dump_spmem(ce_sh, cnte_hbm, core, tile)
        _dump_spmem(cv_sh, cntv_hbm, core, tile, VC_R // NS, VC_R)

    return k(vidx2d, eidx2d)


def _sc_segment_sum(table, gidx2d, sidx2d, split_by_half):
    mesh = plsc.VectorSubcoreMesh(core_axis_name="c", subcore_axis_name="s")
    nslab = 2 if split_by_half else 1

    @functools.partial(
        pl.kernel,
        out_type=jax.ShapeDtypeStruct((NC * ACC_R, SD), jnp.float32),
        mesh=mesh,
        compiler_params=pltpu.CompilerParams(use_tc_tiling_on_sc=False),
        scratch_types=[
            pltpu.VMEM((SS, CHUNK), jnp.int32),
            pltpu.VMEM((SS, CHUNK), jnp.int32),
            [pltpu.VMEM((CHUNK, SD), jnp.float32)] * NB,
            pltpu.VMEM_SHARED((ACC_R, SD), jnp.float32),
            pltpu.SemaphoreType.DMA,
            pltpu.SemaphoreType.DMA,
        ],
    )
    def k(table_hbm, gidx_hbm, sidx_hbm, sum_hbm,
          gi_v, si_v, bufs, acc_sh, semg, sems):
        core = lax.axis_index("c")
        tile = lax.axis_index("s")

        _zero_fill(bufs[0], SD)
        _init_spmem(acc_sh, bufs[0], tile)
        plsc.subcore_barrier()

        def gather(j, b):
            pltpu.async_copy(table_hbm.at[gi_v.at[j]], bufs[b], semg)

        def wait_g(j, b):
            pltpu.make_async_copy(table_hbm.at[gi_v.at[j]], bufs[b], semg).wait()

        def scat(j, b):
            pltpu.async_copy(bufs[b], acc_sh.at[si_v.at[j]], sems, add=True)

        def wait_s(j, b):
            pltpu.make_async_copy(bufs[b], acc_sh.at[si_v.at[j]], sems).wait()

        if split_by_half:
            base0 = tile * nslab * WCH
        else:
            base0 = (core * NS + tile) * WCH
        nsub = nslab * WCH // SS

        def sub_slab(s, _):
            base = base0 + s * SS
            pltpu.sync_copy(gidx_hbm.at[pl.ds(base, SS)], gi_v)
            pltpu.sync_copy(sidx_hbm.at[pl.ds(base, SS)], si_v)
            if split_by_half:
                _remap_slab(si_v, core, SS)

            gather(0, 0)
            for j in range(SS):
                if j + 1 < SS:
                    if j >= 1:
                        wait_s(j - 1, (j + 1) % NB)
                    gather(j + 1, (j + 1) % NB)
                wait_g(j, j % NB)
                scat(j, j % NB)
            wait_s(SS - 2, (SS - 2) % NB)
            wait_s(SS - 1, (SS - 1) % NB)
            return 0

        lax.fori_loop(0, nsub, sub_slab, 0)
        plsc.subcore_barrier()
        _dump_spmem(acc_sh, sum_hbm, core, tile)

    return k(table, gidx2d, sidx2d)


def _tc_gumbel_softmax(ne, gu):
    BR = 400

    def body(ne_ref, gu_ref, out_ref):
        g = -jnp.log(-jnp.log(gu_ref[...] + EPS) + EPS)
        x = (ne_ref[...] + g) / TAU
        for grp in range(S):
            sl = slice(grp * D, (grp + 1) * D)
            xg = x[:, sl]
            m = jnp.max(xg, axis=1, keepdims=True)
            e = jnp.exp(xg - m)
            out_ref[:, sl] = e / jnp.sum(e, axis=1, keepdims=True)

    nblk = V // BR
    return pl.pallas_call(
        body,
        grid=(nblk + 1,),
        in_specs=[
            pl.BlockSpec((BR, SD), lambda i: (jnp.minimum(i, nblk - 1), 0)),
            pl.BlockSpec((BR, SD), lambda i: (jnp.minimum(i, nblk - 1), 0)),
        ],
        out_specs=pl.BlockSpec((BR, SD), lambda i: (i, 0)),
        out_shape=jax.ShapeDtypeStruct((XP_R, SD), jnp.float32),
    )(ne, gu)


def _stats_body(x, ent_ref, psum_ref, n2_ref, g_ref, first):
    @pl.when(first)
    def _():
        ent_ref[...] = jnp.zeros_like(ent_ref)
        psum_ref[...] = jnp.zeros_like(psum_ref)
        n2_ref[...] = jnp.zeros_like(n2_ref)
        g_ref[...] = jnp.zeros_like(g_ref)

    ent_ref[...] += jnp.sum(-x * jnp.log(x + EPS))
    psum_ref[...] += jnp.sum(x, axis=0, keepdims=True)
    n2_ref[...] += jnp.sum(x * x, axis=0, keepdims=True)
    for s in range(S):
        xs = x[:, s * D:(s + 1) * D]
        g_ref[s * D:(s + 1) * D, :] += lax.dot_general(
            xs, xs, (((0,), (0,)), ((), ())),
            preferred_element_type=jnp.float32)


def _tc_stats_y(psum, pcnt):
    BR = 200
    nblk = E // BR
    off = ACC_R // BR

    def body(p0_ref, p1_ref, c0_ref, c1_ref,
             yj_ref, ent_ref, psum_ref, n2_ref, g_ref):
        i = pl.program_id(0)
        cnt = c0_ref[:, 0:1] + c1_ref[:, 0:1]
        yj = (p0_ref[...] + p1_ref[...]) / jnp.maximum(cnt, 1.0)
        yj_ref[...] = yj

        @pl.when(i < nblk)
        def _():
            _stats_body(yj, ent_ref, psum_ref, n2_ref, g_ref, i == 0)

    return pl.pallas_call(
        body,
        grid=(nblk + 1,),
        in_specs=[
            pl.BlockSpec((BR, SD), lambda i: (jnp.minimum(i, nblk - 1), 0)),
            pl.BlockSpec((BR, SD), lambda i: (jnp.minimum(i, nblk - 1) + off, 0)),
            pl.BlockSpec((BR, 16), lambda i: (jnp.minimum(i, nblk - 1), 0)),
            pl.BlockSpec((BR, 16), lambda i: (jnp.minimum(i, nblk - 1) + off, 0)),
        ],
        out_specs=[
            pl.BlockSpec((BR, SD), lambda i: (i, 0)),
            pl.BlockSpec((1, 1), lambda i: (0, 0)),
            pl.BlockSpec((1, SD), lambda i: (0, 0)),
            pl.BlockSpec((1, SD), lambda i: (0, 0)),
            pl.BlockSpec((SD, D), lambda i: (0, 0)),
        ],
        out_shape=[
            jax.ShapeDtypeStruct((YP_R, SD), jnp.float32),
            jax.ShapeDtypeStruct((1, 1), jnp.float32),
            jax.ShapeDtypeStruct((1, SD), jnp.float32),
            jax.ShapeDtypeStruct((1, SD), jnp.float32),
            jax.ShapeDtypeStruct((SD, D), jnp.float32),
        ],
    )(psum, psum, pcnt, pcnt)


def _tc_stats_x(psum, pcnt):
    BR = 200
    nblk = V // BR
    skip = ACC_R // BR
    half_blk = HALF // BR

    coff = VC_R // BR

    def rowmap(i):
        return (jnp.where(i < half_blk, i, i + (skip - half_blk)), 0)

    def body(p_ref, c0_ref, c1_ref, ent_ref, psum_ref, n2_ref, g_ref):
        i = pl.program_id(0)
        cnt = c0_ref[:, 0:1] + c1_ref[:, 0:1]
        xj = p_ref[...] / jnp.maximum(cnt, 1.0)
        _stats_body(xj, ent_ref, psum_ref, n2_ref, g_ref, i == 0)

    return pl.pallas_call(
        body,
        grid=(nblk,),
        in_specs=[
            pl.BlockSpec((BR, SD), rowmap),
            pl.BlockSpec((BR, 16), lambda i: (i, 0)),
            pl.BlockSpec((BR, 16), lambda i: (i + coff, 0)),
        ],
        out_specs=[
            pl.BlockSpec((1, 1), lambda i: (0, 0)),
            pl.BlockSpec((1, SD), lambda i: (0, 0)),
            pl.BlockSpec((1, SD), lambda i: (0, 0)),
            pl.BlockSpec((SD, D), lambda i: (0, 0)),
        ],
        out_shape=[
            jax.ShapeDtypeStruct((1, 1), jnp.float32),
            jax.ShapeDtypeStruct((1, SD), jnp.float32),
            jax.ShapeDtypeStruct((1, SD), jnp.float32),
            jax.ShapeDtypeStruct((SD, D), jnp.float32),
        ],
    )(psum, pcnt, pcnt)


def _finish(ent, psum, n2, g, n_rows):
    ent_mean = ent[0, 0] / (n_rows * S)
    p = psum.reshape(S, D) / n_rows
    glob_ent = -jnp.mean(-jnp.sum(p * jnp.log(p + EPS), axis=1))
    norms = jnp.sqrt(n2.reshape(S, D))
    gm = g.reshape(S, D, D)
    den = jnp.maximum(norms, EPS)
    c = gm / (den[:, :, None] * den[:, None, :])
    c = jax.nn.softmax(c, axis=2)
    diag = jnp.diagonal(c, axis1=1, axis2=2)
    disc = jnp.mean(-jnp.log(diag))
    return ent_mean, glob_ent + disc


def kernel(node_embedding, gumbel_u, V_idx, E_idx):
    gu = gumbel_u.reshape(V, SD)
    x0p = _tc_gumbel_softmax(node_embedding, gu)
    pad = NNZ_PAD - NNZ
    vidx2d = jnp.concatenate(
        [V_idx, jnp.full((pad,), V, jnp.int32)]).reshape(NCH, CHUNK)
    eidx2d = jnp.concatenate(
        [E_idx, jnp.full((pad,), E, jnp.int32)]).reshape(NCH, CHUNK)

    ycnt, xcnt = _sc_counts(vidx2d, eidx2d)

    ysum = _sc_segment_sum(x0p, vidx2d, eidx2d, split_by_half=False)
    yjp, ent_y, psum_y, n2_y, g_y = _tc_stats_y(ysum, ycnt)

    xsum = _sc_segment_sum(yjp, eidx2d, vidx2d, split_by_half=True)
    ent_x, psum_x, n2_x, g_x = _tc_stats_x(xsum, xcnt)

    ly, gy = _finish(ent_y, psum_y, n2_y, g_y, E)
    lx, gx = _finish(ent_x, psum_x, n2_x, g_x, V)
    return (ly + lx, gy + gx)


# speedup vs baseline: 5.0968x; 1.0059x over previous
import functools

import jax
import jax.numpy as jnp
from jax import lax
from jax.experimental import pallas as pl
from jax.experimental.pallas import tpu as pltpu
from jax.experimental.pallas import tpu_sc as plsc

EPS = 1e-10
V = 50000
E = 25000
NNZ = 800000
S = 8
D = 8
SD = S * D
TAU = 1.0

NC = 2
NS = 16
NW = NC * NS
CHUNK = 128
WCH = 196
NCH = NW * WCH
NNZ_PAD = NCH * CHUNK
ACC_R = 25600
TILE_R = ACC_R // NS
HALF = V // 2
TRASH = HALF
XP_R = 50400
YP_R = 25200
VC_R = 51200
NB = 2
SS = 28
SSC = 98


def _zero_fill(ref, ncols):
    def body(i, _):
        for k in range(ncols // 16):
            ref[i, pl.ds(k * 16, 16)] = jnp.zeros((16,), jnp.float32)
        return 0
    lax.fori_loop(0, 128, body, 0)


def _ones_fill(ref):
    def body(i, _):
        ref[i, pl.ds(0, 16)] = jnp.ones((16,), jnp.float32)
        return 0
    lax.fori_loop(0, 128, body, 0)


def _init_spmem(acc, zbuf, tile, tile_rows=TILE_R):
    row0 = tile * tile_rows
    nfull = tile_rows // 128
    rem = tile_rows - nfull * 128
    for k in range(nfull):
        pltpu.sync_copy(zbuf, acc.at[pl.ds(row0 + k * 128, 128)])
    if rem:
        pltpu.sync_copy(zbuf.at[pl.ds(0, rem)], acc.at[pl.ds(row0 + nfull * 128, rem)])


def _dump_spmem(acc, out_hbm, core, tile, tile_rows=TILE_R, acc_rows=ACC_R):
    row0 = tile * tile_rows
    out0 = core * acc_rows + row0
    nfull = tile_rows // 128
    rem = tile_rows - nfull * 128
    for k in range(nfull):
        pltpu.sync_copy(acc.at[pl.ds(row0 + k * 128, 128)],
                        out_hbm.at[pl.ds(out0 + k * 128, 128)])
    if rem:
        pltpu.sync_copy(acc.at[pl.ds(row0 + nfull * 128, rem)],
                        out_hbm.at[pl.ds(out0 + nfull * 128, rem)])


def _remap_slab(idx_slab, core, nrows):
    lo = core * HALF

    def body(j, _):
        for k in range(CHUNK // 16):
            sl = pl.ds(k * 16, 16)
            loc = idx_slab[j, sl] - lo
            inr = (loc >= 0) & (loc < HALF)
            idx_slab[j, sl] = jnp.where(inr, loc, TRASH)
        return 0
    lax.fori_loop(0, nrows, body, 0)


def _sc_counts(vidx2d, eidx2d):
    mesh = plsc.VectorSubcoreMesh(core_axis_name="c", subcore_axis_name="s")

    @functools.partial(
        pl.kernel,
        out_type=(
            jax.ShapeDtypeStruct((NC * ACC_R, 16), jnp.float32),
            jax.ShapeDtypeStruct((NC * VC_R, 16), jnp.float32),
        ),
        mesh=mesh,
        compiler_params=pltpu.CompilerParams(use_tc_tiling_on_sc=False),
        scratch_types=[
            pltpu.VMEM((SSC, CHUNK), jnp.int32),
            pltpu.VMEM((SSC, CHUNK), jnp.int32),
            pltpu.VMEM((128, 16), jnp.float32),
            pltpu.VMEM((128, 16), jnp.float32),
            pltpu.VMEM_SHARED((ACC_R, 16), jnp.float32),
            pltpu.VMEM_SHARED((VC_R, 16), jnp.float32),
            pltpu.SemaphoreType.DMA,
        ],
    )
    def k(vidx_hbm, eidx_hbm, cnte_hbm, cntv_hbm,
          ia_v, ib_v, ones_v, z16_v, ce_sh, cv_sh, sem):
        core = lax.axis_index("c")
        tile = lax.axis_index("s")
        _zero_fill(z16_v, 16)
        _ones_fill(ones_v)
        _init_spmem(ce_sh, z16_v, tile)
        _init_spmem(cv_sh, z16_v, tile, VC_R // NS)
        plsc.subcore_barrier()

        def fire(idx_slab, acc):
            def body(j, _):
                pltpu.async_copy(ones_v, acc.at[idx_slab.at[j]], sem, add=True)
                return 0
            lax.fori_loop(0, SSC, body, 0)

        def drain(n, acc):
            def body(j, _):
                pltpu.make_async_copy(ones_v, acc.at[ia_v.at[0]], sem).wait()
                return 0
            lax.fori_loop(0, n, body, 0)

        w = core * NS + tile
        pltpu.sync_copy(eidx_hbm.at[pl.ds(w * WCH, SSC)], ia_v)
        fire(ia_v, ce_sh)
        pltpu.sync_copy(eidx_hbm.at[pl.ds(w * WCH + SSC, SSC)], ib_v)
        fire(ib_v, ce_sh)
        drain(2 * SSC, ce_sh)
        pltpu.sync_copy(vidx_hbm.at[pl.ds(w * WCH, SSC)], ia_v)
        fire(ia_v, cv_sh)
        pltpu.sync_copy(vidx_hbm.at[pl.ds(w * WCH + SSC, SSC)], ib_v)
        fire(ib_v, cv_sh)
        drain(2 * SSC, cv_sh)
        plsc.subcore_barrier()
        _dump_spmem(ce_sh, cnte_hbm, core, tile)
        _dump_spmem(cv_sh, cntv_hbm, core, tile, VC_R // NS, VC_R)

    return k(vidx2d, eidx2d)


def _sc_segment_sum(table, gidx2d, sidx2d, split_by_half):
    mesh = plsc.VectorSubcoreMesh(core_axis_name="c", subcore_axis_name="s")
    nslab = 2 if split_by_half else 1

    @functools.partial(
        pl.kernel,
        out_type=jax.ShapeDtypeStruct((NC * ACC_R, SD), jnp.float32),
        mesh=mesh,
        compiler_params=pltpu.CompilerParams(use_tc_tiling_on_sc=False),
        scratch_types=[
            pltpu.VMEM((SS, CHUNK), jnp.int32),
            pltpu.VMEM((SS, CHUNK), jnp.int32),
            [pltpu.VMEM((CHUNK, SD), jnp.float32)] * NB,
            pltpu.VMEM_SHARED((ACC_R, SD), jnp.float32),
            pltpu.SemaphoreType.DMA,
            pltpu.SemaphoreType.DMA,
        ],
    )
    def k(table_hbm, gidx_hbm, sidx_hbm, sum_hbm,
          gi_v, si_v, bufs, acc_sh, semg, sems):
        core = lax.axis_index("c")
        tile = lax.axis_index("s")

        _zero_fill(bufs[0], SD)
        _init_spmem(acc_sh, bufs[0], tile)
        plsc.subcore_barrier()

        def gather(j, b):
            pltpu.async_copy(table_hbm.at[gi_v.at[j]], bufs[b], semg)

        def wait_g(j, b):
            pltpu.make_async_copy(table_hbm.at[gi_v.at[j]], bufs[b], semg).wait()

        def scat(j, b):
            pltpu.async_copy(bufs[b], acc_sh.at[si_v.at[j]], sems, add=True)

        def wait_s(j, b):
            pltpu.make_async_copy(bufs[b], acc_sh.at[si_v.at[j]], sems).wait()

        if split_by_half:
            base0 = tile * nslab * WCH
        else:
            base0 = (core * NS + tile) * WCH
        nsub = nslab * WCH // SS

        def sub_slab(s, _):
            base = base0 + s * SS
            pltpu.sync_copy(gidx_hbm.at[pl.ds(base, SS)], gi_v)
            pltpu.sync_copy(sidx_hbm.at[pl.ds(base, SS)], si_v)
            if split_by_half:
                _remap_slab(si_v, core, SS)

            gather(0, 0)
            for j in range(SS):
                if j + 1 < SS:
                    if j >= 1:
                        wait_s(j - 1, (j + 1) % NB)
                    gather(j + 1, (j + 1) % NB)
                wait_g(j, j % NB)
                scat(j, j % NB)
            wait_s(SS - 2, (SS - 2) % NB)
            wait_s(SS - 1, (SS - 1) % NB)
            return 0

        lax.fori_loop(0, nsub, sub_slab, 0)
        plsc.subcore_barrier()
        _dump_spmem(acc_sh, sum_hbm, core, tile)

    return k(table, gidx2d, sidx2d)


def _tc_gumbel_softmax(ne, gu):
    BR = 400

    def body(ne_ref, gu_ref, out_ref):
        g = -jnp.log(-jnp.log(gu_ref[...] + EPS) + EPS)
        x = (ne_ref[...] + g) / TAU
        for grp in range(S):
            sl = slice(grp * D, (grp + 1) * D)
            xg = x[:, sl]
            m = jnp.max(xg, axis=1, keepdims=True)
            e = jnp.exp(xg - m)
            out_ref[:, sl] = e / jnp.sum(e, axis=1, keepdims=True)

    nblk = V // BR
    return pl.pallas_call(
        body,
        grid=(nblk + 1,),
        in_specs=[
            pl.BlockSpec((BR, SD), lambda i: (jnp.minimum(i, nblk - 1), 0)),
            pl.BlockSpec((BR, SD), lambda i: (jnp.minimum(i, nblk - 1), 0)),
        ],
        out_specs=pl.BlockSpec((BR, SD), lambda i: (i, 0)),
        out_shape=jax.ShapeDtypeStruct((XP_R, SD), jnp.float32),
    )(ne, gu)


def _stats_body(x, ent_ref, psum_ref, n2_ref, g_ref, first):
    @pl.when(first)
    def _():
        ent_ref[...] = jnp.zeros_like(ent_ref)
        psum_ref[...] = jnp.zeros_like(psum_ref)
        n2_ref[...] = jnp.zeros_like(n2_ref)
        g_ref[...] = jnp.zeros_like(g_ref)

    ent_ref[...] += jnp.sum(-x * jnp.log(x + EPS))
    psum_ref[...] += jnp.sum(x, axis=0, keepdims=True)
    n2_ref[...] += jnp.sum(x * x, axis=0, keepdims=True)
    for s in range(S):
        xs = x[:, s * D:(s + 1) * D]
        g_ref[s * D:(s + 1) * D, :] += lax.dot_general(
            xs, xs, (((0,), (0,)), ((), ())),
            preferred_element_type=jnp.float32)


def _tc_stats_y(psum, pcnt):
    BR = 200
    nblk = E // BR
    off = ACC_R // BR

    def body(p0_ref, p1_ref, c0_ref, c1_ref,
             yj_ref, ent_ref, psum_ref, n2_ref, g_ref):
        i = pl.program_id(0)
        cnt = c0_ref[:, 0:1] + c1_ref[:, 0:1]
        yj = (p0_ref[...] + p1_ref[...]) / jnp.maximum(cnt, 1.0)
        yj_ref[...] = yj

        @pl.when(i < nblk)
        def _():
            _stats_body(yj, ent_ref, psum_ref, n2_ref, g_ref, i == 0)

    return pl.pallas_call(
        body,
        grid=(nblk + 1,),
        in_specs=[
            pl.BlockSpec((BR, SD), lambda i: (jnp.minimum(i, nblk - 1), 0)),
            pl.BlockSpec((BR, SD), lambda i: (jnp.minimum(i, nblk - 1) + off, 0)),
            pl.BlockSpec((BR, 16), lambda i: (jnp.minimum(i, nblk - 1), 0)),
            pl.BlockSpec((BR, 16), lambda i: (jnp.minimum(i, nblk - 1) + off, 0)),
        ],
        out_specs=[
            pl.BlockSpec((BR, SD), lambda i: (i, 0)),
            pl.BlockSpec((1, 1), lambda i: (0, 0)),
            pl.BlockSpec((1, SD), lambda i: (0, 0)),
            pl.BlockSpec((1, SD), lambda i: (0, 0)),
            pl.BlockSpec((SD, D), lambda i: (0, 0)),
        ],
        out_shape=[
            jax.ShapeDtypeStruct((YP_R, SD), jnp.float32),
            jax.ShapeDtypeStruct((1, 1), jnp.float32),
            jax.ShapeDtypeStruct((1, SD), jnp.float32),
            jax.ShapeDtypeStruct((1, SD), jnp.float32),
            jax.ShapeDtypeStruct((SD, D), jnp.float32),
        ],
    )(psum, psum, pcnt, pcnt)


def _tc_stats_x(psum, pcnt):
    BR = 200
    nblk = V // BR
    skip = ACC_R // BR
    half_blk = HALF // BR

    coff = VC_R // BR

    def rowmap(i):
        return (jnp.where(i < half_blk, i, i + (skip - half_blk)), 0)

    def body(p_ref, c0_ref, c1_ref, ent_ref, psum_ref, n2_ref, g_ref):
        i = pl.program_id(0)
        cnt = c0_ref[:, 0:1] + c1_ref[:, 0:1]
        xj = p_ref[...] / jnp.maximum(cnt, 1.0)
        _stats_body(xj, ent_ref, psum_ref, n2_ref, g_ref, i == 0)

    return pl.pallas_call(
        body,
        grid=(nblk,),
        in_specs=[
            pl.BlockSpec((BR, SD), rowmap),
            pl.BlockSpec((BR, 16), lambda i: (i, 0)),
            pl.BlockSpec((BR, 16), lambda i: (i + coff, 0)),
        ],
        out_specs=[
            pl.BlockSpec((1, 1), lambda i: (0, 0)),
            pl.BlockSpec((1, SD), lambda i: (0, 0)),
            pl.BlockSpec((1, SD), lambda i: (0, 0)),
            pl.BlockSpec((SD, D), lambda i: (0, 0)),
        ],
        out_shape=[
            jax.ShapeDtypeStruct((1, 1), jnp.float32),
            jax.ShapeDtypeStruct((1, SD), jnp.float32),
            jax.ShapeDtypeStruct((1, SD), jnp.float32),
            jax.ShapeDtypeStruct((SD, D), jnp.float32),
        ],
    )(psum, pcnt, pcnt)


def _finish(ent, psum, n2, g, n_rows):
    ent_mean = ent[0, 0] / (n_rows * S)
    p = psum.reshape(S, D) / n_rows
    glob_ent = -jnp.mean(-jnp.sum(p * jnp.log(p + EPS), axis=1))
    norms = jnp.sqrt(n2.reshape(S, D))
    gm = g.reshape(S, D, D)
    den = jnp.maximum(norms, EPS)
    c = gm / (den[:, :, None] * den[:, None, :])
    c = jax.nn.softmax(c, axis=2)
    diag = jnp.diagonal(c, axis1=1, axis2=2)
    disc = jnp.mean(-jnp.log(diag))
    return ent_mean, glob_ent + disc


def kernel(node_embedding, gumbel_u, V_idx, E_idx):
    gu = gumbel_u.reshape(V, SD)
    x0p = _tc_gumbel_softmax(node_embedding, gu)
    pad = NNZ_PAD - NNZ
    vidx2d = jnp.concatenate(
        [V_idx, jnp.full((pad,), V, jnp.int32)]).reshape(NCH, CHUNK)
    eidx2d = jnp.concatenate(
        [E_idx, jnp.full((pad,), E, jnp.int32)]).reshape(NCH, CHUNK)

    ycnt, xcnt = _sc_counts(vidx2d, eidx2d)

    ysum = _sc_segment_sum(x0p, vidx2d, eidx2d, split_by_half=False)
    yjp, ent_y, psum_y, n2_y, g_y = _tc_stats_y(ysum, ycnt)

    xsum = _sc_segment_sum(yjp, eidx2d, vidx2d, split_by_half=True)
    ent_x, psum_x, n2_x, g_x = _tc_stats_x(xsum, xcnt)

    ly, gy = _finish(ent_y, psum_y, n2_y, g_y, E)
    lx, gx = _finish(ent_x, psum_x, n2_x, g_x, V)
    return (ly + lx, gy + gx)
